# Initial kernel scaffold; baseline (speedup 1.0000x reference)
import jax
import jax.numpy as jnp
from jax.experimental import pallas as pl


def kernel(edge_attr, W0, W1, W2):
    raise NotImplementedError("write your pallas kernel here")


# speedup vs baseline: 1.0120x; 1.0120x over previous
import functools

import jax
import jax.numpy as jnp
from jax import lax
from jax.experimental import pallas as pl
from jax.experimental.pallas import tpu as pltpu
from jax.experimental.pallas import tpu_sc as plsc

EMB = 128
NV0, NV1, NV2 = 5, 6, 2
NT = NV0 * NV1 * NV2
N_E = 320000

NC, NS = 2, 16
NW = NC * NS
PER_W = N_E // NW
CHUNK = 400
NCHUNK = PER_W // CHUNK
GRP = 80
NGRP = CHUNK // GRP

EA_COLS = 3 * EMB
EA_ROWS = N_E // EMB


def _prep_body(ea_ref, w0_ref, w1_ref, w2_ref, t_ref, code_ref):
    r = lax.broadcasted_iota(jnp.int32, (NT, 1), 0)
    oh0 = (r // (NV1 * NV2) == lax.broadcasted_iota(jnp.int32, (NT, NV0), 1))
    oh1 = ((r // NV2) % NV1 == lax.broadcasted_iota(jnp.int32, (NT, NV1), 1))
    oh2 = (r % NV2 == lax.broadcasted_iota(jnp.int32, (NT, NV2), 1))
    t = jnp.dot(oh0.astype(jnp.float32), w0_ref[...],
                preferred_element_type=jnp.float32)
    t = t + jnp.dot(oh1.astype(jnp.float32), w1_ref[...],
                    preferred_element_type=jnp.float32)
    t = t + jnp.dot(oh2.astype(jnp.float32), w2_ref[...],
                    preferred_element_type=jnp.float32)
    t_ref[...] = t

    f = lax.broadcasted_iota(jnp.int32, (EA_COLS, EMB), 0)
    e3 = lax.broadcasted_iota(jnp.int32, (EA_COLS, EMB), 1) * 3
    s = (jnp.where(f == e3, 12.0, 0.0) + jnp.where(f == e3 + 1, 2.0, 0.0)
         + jnp.where(f == e3 + 2, 1.0, 0.0))
    codes = jnp.dot(ea_ref[...].astype(jnp.float32), s,
                    preferred_element_type=jnp.float32)
    code_ref[...] = codes.astype(jnp.int32)


def _prep(ea2, w0, w1, w2):
    return pl.pallas_call(
        _prep_body,
        out_shape=(jax.ShapeDtypeStruct((NT, EMB), jnp.float32),
                   jax.ShapeDtypeStruct((EA_ROWS, EMB), jnp.int32)),
    )(ea2, w0, w1, w2)


_mesh = plsc.VectorSubcoreMesh(core_axis_name="c", subcore_axis_name="s")


@functools.partial(
    pl.kernel,
    mesh=_mesh,
    out_type=jax.ShapeDtypeStruct((N_E, EMB), jnp.float32),
    scratch_types=[
        pltpu.VMEM((CHUNK,), jnp.int32),
        pltpu.VMEM((CHUNK, EMB), jnp.float32),
        pltpu.SemaphoreType.DMA,
    ],
)
def _gather_kernel(code_hbm, t_hbm, out_hbm, code_v, rows_v, sem):
    wid = lax.axis_index("s") * NC + lax.axis_index("c")

    def chunk_body(g, carry):
        base = wid * PER_W + g * CHUNK
        pltpu.sync_copy(code_hbm.at[pl.ds(base, CHUNK)], code_v)
        copies = []
        for j in range(NGRP):
            copies.append(pltpu.async_copy(
                t_hbm.at[code_v.at[pl.ds(j * GRP, GRP)]],
                rows_v.at[pl.ds(j * GRP, GRP)], sem))
        for cp in copies:
            cp.wait()
        pltpu.sync_copy(rows_v, out_hbm.at[pl.ds(base, CHUNK)])
        return carry

    lax.fori_loop(0, NCHUNK, chunk_body, 0)


def kernel(edge_attr, W0, W1, W2):
    ea2 = edge_attr.astype(jnp.int32).reshape(EA_ROWS, EA_COLS)
    t, codes = _prep(ea2, W0, W1, W2)
    return _gather_kernel(codes.reshape(N_E), t)


# speedup vs baseline: 7.5794x; 7.4895x over previous
import functools

import jax
import jax.numpy as jnp
from jax import lax
from jax.experimental import pallas as pl
from jax.experimental.pallas import tpu as pltpu
from jax.experimental.pallas import tpu_sc as plsc

EMB = 128
NV0, NV1, NV2 = 5, 6, 2
NT = NV0 * NV1 * NV2
N_E = 320000

NC, NS = 2, 16
NW = NC * NS
PER_W = N_E // NW
CHUNK = 400
NCHUNK = PER_W // CHUNK
GRP = 80
NGRP = CHUNK // GRP

EA_COLS = 3 * EMB
EA_ROWS = N_E // EMB


def _prep_body(ea_ref, w0_ref, w1_ref, w2_ref, t_ref, code_ref):
    r = lax.broadcasted_iota(jnp.int32, (NT, 1), 0)
    oh0 = (r // (NV1 * NV2) == lax.broadcasted_iota(jnp.int32, (NT, NV0), 1))
    oh1 = ((r // NV2) % NV1 == lax.broadcasted_iota(jnp.int32, (NT, NV1), 1))
    oh2 = (r % NV2 == lax.broadcasted_iota(jnp.int32, (NT, NV2), 1))
    t = jnp.dot(oh0.astype(jnp.float32), w0_ref[...],
                preferred_element_type=jnp.float32)
    t = t + jnp.dot(oh1.astype(jnp.float32), w1_ref[...],
                    preferred_element_type=jnp.float32)
    t = t + jnp.dot(oh2.astype(jnp.float32), w2_ref[...],
                    preferred_element_type=jnp.float32)
    t_ref[...] = t

    f = lax.broadcasted_iota(jnp.int32, (EA_COLS, EMB), 0)
    e3 = lax.broadcasted_iota(jnp.int32, (EA_COLS, EMB), 1) * 3
    s = (jnp.where(f == e3, 12.0, 0.0) + jnp.where(f == e3 + 1, 2.0, 0.0)
         + jnp.where(f == e3 + 2, 1.0, 0.0))
    codes = jnp.dot(ea_ref[...].astype(jnp.float32), s,
                    preferred_element_type=jnp.float32)
    code_ref[...] = codes.astype(jnp.int32)


def _prep(ea2, w0, w1, w2):
    return pl.pallas_call(
        _prep_body,
        out_shape=(jax.ShapeDtypeStruct((NT, EMB), jnp.float32),
                   jax.ShapeDtypeStruct((EA_ROWS, EMB), jnp.int32)),
    )(ea2, w0, w1, w2)


_mesh = plsc.VectorSubcoreMesh(core_axis_name="c", subcore_axis_name="s")


@functools.partial(
    pl.kernel,
    mesh=_mesh,
    out_type=jax.ShapeDtypeStruct((N_E, EMB), jnp.float32),
    scratch_types=[
        pltpu.VMEM_SHARED((NT, EMB), jnp.float32),
        pltpu.VMEM((CHUNK,), jnp.int32),
        pltpu.VMEM((CHUNK,), jnp.int32),
        pltpu.VMEM((CHUNK, EMB), jnp.float32),
        pltpu.VMEM((CHUNK, EMB), jnp.float32),
        pltpu.SemaphoreType.DMA,
        pltpu.SemaphoreType.DMA,
        pltpu.SemaphoreType.DMA,
        pltpu.SemaphoreType.DMA,
    ],
)
def _gather_kernel(code_hbm, t_hbm, out_hbm,
                   t_sh, c0, c1, r0, r1, gsem0, gsem1, ssem0, ssem1):
    sid = lax.axis_index("s")
    wid = sid * NC + lax.axis_index("c")

    @pl.when(sid == 0)
    def _stage_table():
        pltpu.sync_copy(t_hbm, t_sh)

    plsc.subcore_barrier()

    def load_codes(g, cbuf):
        pltpu.sync_copy(code_hbm.at[pl.ds(wid * PER_W + g * CHUNK, CHUNK)],
                        cbuf)

    def fire_gathers(cbuf, rbuf, sem):
        for j in range(NGRP):
            pltpu.async_copy(t_sh.at[cbuf.at[pl.ds(j * GRP, GRP)]],
                             rbuf.at[pl.ds(j * GRP, GRP)], sem)

    def drain_gathers(rbuf, sem):
        pltpu.make_async_copy(out_hbm.at[pl.ds(0, CHUNK)], rbuf, sem).wait()

    def fire_store(g, rbuf, sem):
        pltpu.async_copy(rbuf, out_hbm.at[pl.ds(wid * PER_W + g * CHUNK,
                                                CHUNK)], sem)

    def drain_store(rbuf, sem):
        pltpu.make_async_copy(rbuf, out_hbm.at[pl.ds(0, CHUNK)], sem).wait()

    load_codes(0, c0)
    fire_gathers(c0, r0, gsem0)

    def pair_body(p, carry):
        g0 = 2 * p
        g1 = 2 * p + 1
        g2 = 2 * p + 2

        @pl.when(p > 0)
        def _():
            drain_store(r1, ssem1)
        load_codes(g1, c1)
        fire_gathers(c1, r1, gsem1)

        drain_gathers(r0, gsem0)
        fire_store(g0, r0, ssem0)

        load_codes(g2, c0)
        drain_store(r0, ssem0)
        fire_gathers(c0, r0, gsem0)

        drain_gathers(r1, gsem1)
        fire_store(g1, r1, ssem1)
        return carry

    lax.fori_loop(0, (NCHUNK - 1) // 2, pair_body, 0)

    drain_gathers(r0, gsem0)
    fire_store(NCHUNK - 1, r0, ssem0)
    drain_store(r1, ssem1)
    drain_store(r0, ssem0)


def kernel(edge_attr, W0, W1, W2):
    ea2 = edge_attr.astype(jnp.int32).reshape(EA_ROWS, EA_COLS)
    t, codes = _prep(ea2, W0, W1, W2)
    return _gather_kernel(codes.reshape(N_E), t)
